# baseline (speedup 1.0000x reference)
import jax
import jax.numpy as jnp
from jax.experimental import pallas as pl
from jax.experimental.pallas import tpu as pltpu


def _linear_kernel(fa_ref, fb_ref, w_ref, b_ref, oa_ref, ob_ref):
    w = w_ref[...].astype(jnp.bfloat16)
    bias = b_ref[...]
    dims = (((1,), (1,)), ((), ()))

    def mm(a):
        return jax.lax.dot_general(
            a.astype(jnp.bfloat16), w, dimension_numbers=dims,
            preferred_element_type=jnp.float32,
        )

    oa_ref[...] = mm(fa_ref[...]) + bias
    ob_ref[...] = mm(fb_ref[...]) + bias


def kernel(feat, W, b):
    B, D = feat.shape
    C = W.shape[0]
    Bm = 1024
    half = B // 2
    n = half // Bm
    out_a, out_b = pl.pallas_call(
        _linear_kernel,
        grid=(n,),
        in_specs=[
            pl.BlockSpec((Bm, D), lambda i: (i, 0)),
            pl.BlockSpec((Bm, D), lambda i, _n=n: (i + _n, 0)),
            pl.BlockSpec((C, D), lambda i: (0, 0)),
            pl.BlockSpec((1, C), lambda i: (0, 0)),
        ],
        out_specs=[
            pl.BlockSpec((Bm, C), lambda i: (i, 0)),
            pl.BlockSpec((Bm, C), lambda i: (i, 0)),
        ],
        out_shape=[
            jax.ShapeDtypeStruct((half, C), jnp.float32),
            jax.ShapeDtypeStruct((half, C), jnp.float32),
        ],
        compiler_params=pltpu.CompilerParams(
            dimension_semantics=("parallel",),
        ),
    )(feat, feat, W, b.reshape(1, C))
    return jnp.concatenate([out_a, out_b], axis=0)
